# baseline (speedup 1.0000x reference)
import jax
import jax.numpy as jnp
import numpy as np
from jax import lax
from jax.experimental import pallas as pl
from jax.experimental.pallas import tpu as pltpu
from jax.experimental.pallas import tpu_sc as plsc

_BATCH = 16384
_NFIELD = 26
_EMBED = 16
_NCORES = 2
_NSUB = 16
_NW = _NCORES * _NSUB
_BPW = _BATCH // _NW
_LPW = _BPW * _NFIELD
_FLAT = _BATCH * _NFIELD
_GCHUNK = 512


def _wid():
    return lax.axis_index("s") * _NCORES + lax.axis_index("c")


def _k1_index(xt_hbm, idx_hbm, xv, idxb):
    b0 = _wid() * _BPW
    for f in range(_NFIELD):
        pltpu.sync_copy(xt_hbm.at[f, pl.ds(b0, _BPW)], xv)
        for j in range(_BPW // 16):
            bloc = j * 16 + lax.iota(jnp.int32, 16)
            pos = bloc * _NFIELD + f
            val = xv[pl.ds(j * 16, 16)] + f * 100000
            plsc.store_scatter(idxb, [pos], val)
    pltpu.sync_copy(idxb, idx_hbm.at[pl.ds(_wid() * _LPW, _LPW)])


def _k2_gather(idx_hbm, tab_hbm, rows_hbm, idxv, gidx, rsub, gv, rv1, sem):
    base = _wid() * _LPW
    tabw = tab_hbm

    @pl.loop(0, _LPW // _GCHUNK)
    def _chunk(ci):
        cbase = base + ci * _GCHUNK
        pltpu.sync_copy(idx_hbm.at[pl.ds(cbase, _GCHUNK)], idxv)
        for s in range(_GCHUNK // 16):
            iv = idxv[pl.ds(s * 16, 16)]
            gidx[pl.ds(s * 16, 16)] = lax.shift_right_logical(iv, 3)
            rsub[pl.ds(s * 16, 16)] = jnp.bitwise_and(iv, 7) * _EMBED
        pltpu.async_copy(tabw.at[gidx], gv, sem).wait()

        @pl.loop(0, _GCHUNK // 16)
        def _jstep(j):
            kv = j * 16 + lax.iota(jnp.int32, 16)
            rs = rsub[pl.ds(j * 16, 16)]
            for e in range(_EMBED):
                vals = plsc.load_gather(gv, [kv, rs + e])
                pos = j * 256 + lax.iota(jnp.int32, 16) * _EMBED + e
                plsc.store_scatter(rv1, [pos], vals)

        pltpu.sync_copy(
            rv1, rows_hbm.at[pl.ds(cbase * _EMBED, _GCHUNK * _EMBED)]
        )


_CB3 = 128
_FE = _NFIELD * _EMBED


def _k3_pack(rows_hbm, out_hbm, rbuf, o2):
    b0 = _wid() * _BPW

    @pl.loop(0, _BPW // _CB3)
    def _chunk(ci):
        bb = b0 + ci * _CB3
        pltpu.sync_copy(
            rows_hbm.at[pl.ds(bb * _FE, _CB3 * _FE)], rbuf
        )

        @pl.loop(0, _CB3 // 16)
        def _jstep(j):
            base = j * 16 * _FE + lax.iota(jnp.int32, 16) * _FE
            for fe in range(_FE):
                vals = plsc.load_gather(rbuf, [base + fe])
                o2[fe, pl.ds(j * 16, 16)] = vals

        pltpu.sync_copy(o2, out_hbm.at[:, pl.ds(bb, _CB3)])


def kernel(x, table):
    mesh = plsc.VectorSubcoreMesh(core_axis_name="c", subcore_axis_name="s")
    cp_tc = pltpu.CompilerParams(
        use_tc_tiling_on_sc=True, needs_layout_passes=False
    )
    cp_dense = pltpu.CompilerParams(
        use_tc_tiling_on_sc=False, needs_layout_passes=False
    )

    k1 = pl.kernel(
        _k1_index,
        out_type=jax.ShapeDtypeStruct((_FLAT,), jnp.int32),
        mesh=mesh,
        scratch_types=[
            pltpu.VMEM((_BPW,), jnp.int32),
            pltpu.VMEM((_LPW,), jnp.int32),
        ],
        compiler_params=cp_tc,
    )
    k2 = pl.kernel(
        _k2_gather,
        out_type=jax.ShapeDtypeStruct((_FLAT * _EMBED,), jnp.float32),
        mesh=mesh,
        scratch_types=[
            pltpu.VMEM((_GCHUNK,), jnp.int32),
            pltpu.VMEM((_GCHUNK,), jnp.int32),
            pltpu.VMEM((_GCHUNK,), jnp.int32),
            pltpu.VMEM((_GCHUNK, 8 * _EMBED), jnp.float32),
            pltpu.VMEM((_GCHUNK * _EMBED,), jnp.float32),
            pltpu.SemaphoreType.DMA,
        ],
        compiler_params=cp_tc,
    )
    k3 = pl.kernel(
        _k3_pack,
        out_type=jax.ShapeDtypeStruct((_FE, _BATCH), jnp.float32),
        mesh=mesh,
        scratch_types=[
            pltpu.VMEM((_CB3 * _FE,), jnp.float32),
            pltpu.VMEM((_FE, _CB3), jnp.float32),
        ],
        compiler_params=cp_tc,
    )

    idx = k1(x.T)
    rows = k2(idx, table.reshape(2600000 // 8, 8 * _EMBED))
    outp = k3(rows)
    return outp.reshape(_NFIELD, _EMBED, _BATCH).transpose(2, 0, 1)


# speedup vs baseline: 1.3133x; 1.3133x over previous
import jax
import jax.numpy as jnp
from jax import lax
from jax.experimental import pallas as pl
from jax.experimental.pallas import tpu as pltpu
from jax.experimental.pallas import tpu_sc as plsc

_BATCH = 16384
_NFIELD = 26
_EMBED = 16
_NROW = 2600000
_NCORES = 2
_NSUB = 16
_NW = _NCORES * _NSUB
_BPW = _BATCH // _NW
_LPW = _BPW * _NFIELD
_FLAT = _BATCH * _NFIELD
_GCHUNK = 512
_NBLK = _NROW // 128
_BLK_PER_W = (_NBLK + _NW - 1) // _NW
_CB3 = 128
_FE = _NFIELD * _EMBED


def _wid():
    return lax.axis_index("s") * _NCORES + lax.axis_index("c")


def _transpose_block(blkv, obuf, j):
    rv = j * 16 + lax.iota(jnp.int32, 16)
    for e in range(_EMBED):
        ev = lax.iota(jnp.int32, 16) * 0 + e
        vals = plsc.load_gather(blkv, [ev, rv])
        plsc.store_scatter(obuf, [rv * _EMBED + e], vals)


def _k0_detile(tabt_hbm, tail_hbm, tabrm_hbm, blkv, obuf, tailb):
    w = _wid()

    @pl.loop(0, _BLK_PER_W)
    def _blk(i):
        blk = w * _BLK_PER_W + i

        @pl.when(blk < _NBLK)
        def _():
            pltpu.sync_copy(tabt_hbm.at[:, pl.ds(blk * 128, 128)], blkv)

            @pl.loop(0, 8)
            def _jstep(j):
                _transpose_block(blkv, obuf, j)

            pltpu.sync_copy(obuf, tabrm_hbm.at[pl.ds(blk * 2048, 2048)])

    @pl.when(w == _NW - 1)
    def _tail():
        pltpu.sync_copy(tail_hbm, tailb)
        pltpu.sync_copy(
            tailb, tabrm_hbm.at[pl.ds(_NBLK * 128 * _EMBED, 1024)]
        )


def _k1_index(xt_hbm, idx_hbm, xv, idxb):
    b0 = _wid() * _BPW
    for f in range(_NFIELD):
        pltpu.sync_copy(xt_hbm.at[f, pl.ds(b0, _BPW)], xv)
        for j in range(_BPW // 16):
            bloc = j * 16 + lax.iota(jnp.int32, 16)
            pos = bloc * _NFIELD + f
            val = xv[pl.ds(j * 16, 16)] + f * 100000
            plsc.store_scatter(idxb, [pos], val)
    pltpu.sync_copy(idxb, idx_hbm.at[pl.ds(_wid() * _LPW, _LPW)])


def _k2_gather(idx_hbm, tab_hbm, rows_hbm, idxv, rv, rv1, sem):
    base = _wid() * _LPW

    @pl.loop(0, _LPW // _GCHUNK)
    def _chunk(ci):
        cbase = base + ci * _GCHUNK
        pltpu.sync_copy(idx_hbm.at[pl.ds(cbase, _GCHUNK)], idxv)
        pltpu.async_copy(tab_hbm.at[idxv], rv, sem).wait()

        @pl.loop(0, _GCHUNK, unroll=8)
        def _row(r):
            rv1[pl.ds(r * _EMBED, _EMBED)] = rv[r]

        pltpu.sync_copy(
            rv1, rows_hbm.at[pl.ds(cbase * _EMBED, _GCHUNK * _EMBED)]
        )


def _k3_pack(rows_hbm, out_hbm, rbuf, o2):
    b0 = _wid() * _BPW

    @pl.loop(0, _BPW // _CB3)
    def _chunk(ci):
        bb = b0 + ci * _CB3
        pltpu.sync_copy(rows_hbm.at[pl.ds(bb * _FE, _CB3 * _FE)], rbuf)

        @pl.loop(0, _CB3 // 16)
        def _jstep(j):
            base = j * 16 * _FE + lax.iota(jnp.int32, 16) * _FE
            for fe in range(_FE):
                vals = plsc.load_gather(rbuf, [base + fe])
                o2[fe, pl.ds(j * 16, 16)] = vals

        pltpu.sync_copy(o2, out_hbm.at[:, pl.ds(bb, _CB3)])


def kernel(x, table):
    mesh = plsc.VectorSubcoreMesh(core_axis_name="c", subcore_axis_name="s")
    cp_tc = pltpu.CompilerParams(
        use_tc_tiling_on_sc=True, needs_layout_passes=False
    )
    cp_dense = pltpu.CompilerParams(
        use_tc_tiling_on_sc=False, needs_layout_passes=False
    )

    k0 = pl.kernel(
        _k0_detile,
        out_type=jax.ShapeDtypeStruct((_NROW * _EMBED,), jnp.float32),
        mesh=mesh,
        scratch_types=[
            pltpu.VMEM((_EMBED, 128), jnp.float32),
            pltpu.VMEM((2048,), jnp.float32),
            pltpu.VMEM((1024,), jnp.float32),
        ],
        compiler_params=cp_tc,
    )
    k1 = pl.kernel(
        _k1_index,
        out_type=jax.ShapeDtypeStruct((_FLAT,), jnp.int32),
        mesh=mesh,
        scratch_types=[
            pltpu.VMEM((_BPW,), jnp.int32),
            pltpu.VMEM((_LPW,), jnp.int32),
        ],
        compiler_params=cp_tc,
    )
    k2 = pl.kernel(
        _k2_gather,
        out_type=jax.ShapeDtypeStruct((_FLAT * _EMBED,), jnp.float32),
        mesh=mesh,
        scratch_types=[
            pltpu.VMEM((_GCHUNK,), jnp.int32),
            pltpu.VMEM((_GCHUNK, _EMBED), jnp.float32),
            pltpu.VMEM((_GCHUNK * _EMBED,), jnp.float32),
            pltpu.SemaphoreType.DMA,
        ],
        compiler_params=cp_dense,
    )
    k3 = pl.kernel(
        _k3_pack,
        out_type=jax.ShapeDtypeStruct((_FE, _BATCH), jnp.float32),
        mesh=mesh,
        scratch_types=[
            pltpu.VMEM((_CB3 * _FE,), jnp.float32),
            pltpu.VMEM((_FE, _CB3), jnp.float32),
        ],
        compiler_params=cp_tc,
    )

    idx = k1(x.T)
    tail = table[_NBLK * 128 :].reshape(64 * _EMBED)
    tabrm = k0(table.T, tail)
    rows = k2(idx, tabrm.reshape(_NROW, _EMBED))
    outp = k3(rows)
    return outp.reshape(_NFIELD, _EMBED, _BATCH).transpose(2, 0, 1)
